# baseline (speedup 1.0000x reference)
import functools

import jax
import jax.numpy as jnp
from jax import lax
from jax.experimental import pallas as pl
from jax.experimental.pallas import tpu as pltpu
from jax.experimental.pallas import tpu_sc as plsc

BATCH = 16384
EMBED = 64
HIDDEN = 256
NROWS = 1000000

NC = 2
NS = 16
NW = NC * NS
B_PER_W = BATCH // NW

PAIRS = 7813
P_ROWS = PAIRS * EMBED
PAIRS_PER_STEP = 13
PACK_STEPS = 601


def _pack_body(x_ref, o_ref):
    x = x_ref[...]
    for k in range(PAIRS_PER_STEP):
        lo = x[:, 128 * k:128 * k + 64].T
        hi = x[:, 128 * k + 64:128 * k + 128].T
        o_ref[64 * k:64 * k + 64, :] = jnp.concatenate([lo, hi], axis=1)


def _tc_pack(tabT):
    return pl.pallas_call(
        _pack_body,
        grid=(PACK_STEPS,),
        in_specs=[pl.BlockSpec((EMBED, PAIRS_PER_STEP * 128),
                               lambda g: (0, g))],
        out_specs=pl.BlockSpec((PAIRS_PER_STEP * EMBED, 128),
                               lambda g: (g, 0)),
        out_shape=jax.ShapeDtypeStruct((P_ROWS, 128), jnp.float32),
    )(tabT)


def _sc_gather(ptab, idx):
    mesh = plsc.VectorSubcoreMesh(core_axis_name="c", subcore_axis_name="s")

    @functools.partial(
        pl.kernel,
        mesh=mesh,
        out_type=jax.ShapeDtypeStruct((BATCH, 128), jnp.float32),
        scratch_types=[
            pltpu.VMEM((B_PER_W,), jnp.int32),
            pltpu.VMEM((B_PER_W, 128), jnp.float32),
            pltpu.SemaphoreType.DMA,
        ],
    )
    def gather_kernel(tab_hbm, id_hbm, out_hbm, idx_v, rows_v, sem):
        wid = lax.axis_index("s") * NC + lax.axis_index("c")
        base = wid * B_PER_W
        pltpu.sync_copy(id_hbm.at[pl.ds(base, B_PER_W)], idx_v)
        pltpu.async_copy(tab_hbm.at[idx_v], rows_v, sem).wait()
        pltpu.sync_copy(rows_v, out_hbm.at[pl.ds(base, B_PER_W)])

    return gather_kernel(ptab, idx)


def _mlp_kernel(gu_ref, gi_ref, pu_ref, pi_ref, w1u_ref, w1i_ref, b1_ref,
                w2_ref, b2_ref, o_ref):
    gu = gu_ref[...]
    gi = gi_ref[...]
    u = jnp.where(pu_ref[...] == 0, gu[:, :EMBED], gu[:, EMBED:])
    i = jnp.where(pi_ref[...] == 0, gi[:, :EMBED], gi[:, EMBED:])
    h = (
        jnp.dot(u, w1u_ref[...], preferred_element_type=jnp.float32)
        + jnp.dot(i, w1i_ref[...], preferred_element_type=jnp.float32)
        + b1_ref[...]
    )
    h = jnp.where(h >= 0, h, 0.2 * h)
    out = jnp.dot(h, w2_ref[...], preferred_element_type=jnp.float32) \
        + b2_ref[...]
    o_ref[...] = jax.nn.sigmoid(out)


def _tc_mlp(gu, gi, pu, pi, W1u, W1i, b1, W2, b2):
    blk = 2048
    grid = (BATCH // blk,)
    return pl.pallas_call(
        _mlp_kernel,
        grid=grid,
        in_specs=[
            pl.BlockSpec((blk, 128), lambda g: (g, 0)),
            pl.BlockSpec((blk, 128), lambda g: (g, 0)),
            pl.BlockSpec((blk, 1), lambda g: (g, 0)),
            pl.BlockSpec((blk, 1), lambda g: (g, 0)),
            pl.BlockSpec((EMBED, HIDDEN), lambda g: (0, 0)),
            pl.BlockSpec((EMBED, HIDDEN), lambda g: (0, 0)),
            pl.BlockSpec((1, HIDDEN), lambda g: (0, 0)),
            pl.BlockSpec((HIDDEN, 1), lambda g: (0, 0)),
            pl.BlockSpec((1, 1), lambda g: (0, 0)),
        ],
        out_specs=pl.BlockSpec((blk, 1), lambda g: (g, 0)),
        out_shape=jax.ShapeDtypeStruct((BATCH, 1), jnp.float32),
    )(gu, gi, pu, pi, W1u, W1i, b1, W2, b2)


def kernel(user_ids, item_ids, user_table, item_table, W1, b1, W2, b2):
    uid = user_ids.astype(jnp.int32)
    iid = item_ids.astype(jnp.int32)
    urow = (uid // 128) * EMBED + uid % EMBED
    irow = (iid // 128) * EMBED + iid % EMBED
    uhalf = ((uid // EMBED) % 2).reshape(BATCH, 1)
    ihalf = ((iid // EMBED) % 2).reshape(BATCH, 1)
    pu_tab = _tc_pack(user_table.T)
    gu = _sc_gather(pu_tab, urow)
    pi_tab = _tc_pack(item_table.T)
    gi = _sc_gather(pi_tab, irow)
    W1u = W1[:EMBED]
    W1i = W1[EMBED:]
    return _tc_mlp(gu, gi, uhalf, ihalf, W1u, W1i, b1.reshape(1, HIDDEN), W2,
                   b2.reshape(1, 1))


# speedup vs baseline: 1.3871x; 1.3871x over previous
import functools

import jax
import jax.numpy as jnp
from jax import lax
from jax.experimental import pallas as pl
from jax.experimental.pallas import tpu as pltpu
from jax.experimental.pallas import tpu_sc as plsc

BATCH = 16384
EMBED = 64
HIDDEN = 256
NROWS = 1000000

NC = 2
NS = 16
NW = NC * NS
B_PER_W = BATCH // NW

P_ROWS = 500224
PACK_W = 512
PACK_STEPS = 977


def _pack_body(xl_ref, xr_ref, o_ref):
    o_ref[:, :EMBED] = xl_ref[...].T
    o_ref[:, EMBED:] = xr_ref[...].T


def _tc_pack(tabT):
    return pl.pallas_call(
        _pack_body,
        grid=(PACK_STEPS,),
        in_specs=[
            pl.BlockSpec((EMBED, PACK_W), lambda g: (0, g)),
            pl.BlockSpec((EMBED, PACK_W), lambda g: (0, g + PACK_STEPS)),
        ],
        out_specs=pl.BlockSpec((PACK_W, 128), lambda g: (g, 0)),
        out_shape=jax.ShapeDtypeStruct((P_ROWS, 128), jnp.float32),
    )(tabT, tabT)


def _sc_gather(ptab, idx):
    mesh = plsc.VectorSubcoreMesh(core_axis_name="c", subcore_axis_name="s")

    @functools.partial(
        pl.kernel,
        mesh=mesh,
        out_type=jax.ShapeDtypeStruct((BATCH, 128), jnp.float32),
        scratch_types=[
            pltpu.VMEM((B_PER_W,), jnp.int32),
            pltpu.VMEM((B_PER_W, 128), jnp.float32),
            pltpu.SemaphoreType.DMA,
        ],
    )
    def gather_kernel(tab_hbm, id_hbm, out_hbm, idx_v, rows_v, sem):
        wid = lax.axis_index("s") * NC + lax.axis_index("c")
        base = wid * B_PER_W
        pltpu.sync_copy(id_hbm.at[pl.ds(base, B_PER_W)], idx_v)
        pltpu.async_copy(tab_hbm.at[idx_v], rows_v, sem).wait()
        pltpu.sync_copy(rows_v, out_hbm.at[pl.ds(base, B_PER_W)])

    return gather_kernel(ptab, idx)


def _mlp_kernel(gu_ref, gi_ref, pu_ref, pi_ref, w1u_ref, w1i_ref, b1_ref,
                w2_ref, b2_ref, o_ref):
    gu = gu_ref[...]
    gi = gi_ref[...]
    u = jnp.where(pu_ref[...] == 0, gu[:, :EMBED], gu[:, EMBED:])
    i = jnp.where(pi_ref[...] == 0, gi[:, :EMBED], gi[:, EMBED:])
    h = (
        jnp.dot(u, w1u_ref[...], preferred_element_type=jnp.float32)
        + jnp.dot(i, w1i_ref[...], preferred_element_type=jnp.float32)
        + b1_ref[...]
    )
    h = jnp.where(h >= 0, h, 0.2 * h)
    out = jnp.dot(h, w2_ref[...], preferred_element_type=jnp.float32) \
        + b2_ref[...]
    o_ref[...] = jax.nn.sigmoid(out)


def _tc_mlp(gu, gi, pu, pi, W1u, W1i, b1, W2, b2):
    blk = 2048
    grid = (BATCH // blk,)
    return pl.pallas_call(
        _mlp_kernel,
        grid=grid,
        in_specs=[
            pl.BlockSpec((blk, 128), lambda g: (g, 0)),
            pl.BlockSpec((blk, 128), lambda g: (g, 0)),
            pl.BlockSpec((blk, 1), lambda g: (g, 0)),
            pl.BlockSpec((blk, 1), lambda g: (g, 0)),
            pl.BlockSpec((EMBED, HIDDEN), lambda g: (0, 0)),
            pl.BlockSpec((EMBED, HIDDEN), lambda g: (0, 0)),
            pl.BlockSpec((1, HIDDEN), lambda g: (0, 0)),
            pl.BlockSpec((HIDDEN, 1), lambda g: (0, 0)),
            pl.BlockSpec((1, 1), lambda g: (0, 0)),
        ],
        out_specs=pl.BlockSpec((blk, 1), lambda g: (g, 0)),
        out_shape=jax.ShapeDtypeStruct((BATCH, 1), jnp.float32),
    )(gu, gi, pu, pi, W1u, W1i, b1, W2, b2)


def kernel(user_ids, item_ids, user_table, item_table, W1, b1, W2, b2):
    uid = user_ids.astype(jnp.int32)
    iid = item_ids.astype(jnp.int32)
    urow = jnp.where(uid >= P_ROWS, uid - P_ROWS, uid)
    irow = jnp.where(iid >= P_ROWS, iid - P_ROWS, iid)
    uhalf = (uid >= P_ROWS).astype(jnp.int32).reshape(BATCH, 1)
    ihalf = (iid >= P_ROWS).astype(jnp.int32).reshape(BATCH, 1)
    pu_tab = _tc_pack(user_table.T)
    gu = _sc_gather(pu_tab, urow)
    pi_tab = _tc_pack(item_table.T)
    gi = _sc_gather(pi_tab, irow)
    W1u = W1[:EMBED]
    W1i = W1[EMBED:]
    return _tc_mlp(gu, gi, uhalf, ihalf, W1u, W1i, b1.reshape(1, HIDDEN), W2,
                   b2.reshape(1, 1))
